# baseline (speedup 1.0000x reference)
import functools

import jax
import jax.numpy as jnp
from jax import lax
from jax.experimental import pallas as pl
from jax.experimental.pallas import tpu as pltpu
from jax.experimental.pallas import tpu_sc as plsc

_NSLOT = 4
_NGATHER = 2


def _make_gather(n_rows: int, dim: int, chunk: int):
    info = plsc.get_sparse_core_info()
    nc, ns = info.num_cores, info.num_subcores
    nw = nc * ns
    assert n_rows % (nw * chunk) == 0
    b_per_w = n_rows // nw
    n_iters = b_per_w // chunk

    mesh = plsc.VectorSubcoreMesh(core_axis_name="c", subcore_axis_name="s")

    @functools.partial(
        pl.kernel,
        mesh=mesh,
        out_type=jax.ShapeDtypeStruct((n_rows, dim), jnp.float32),
        scratch_types=[
            pltpu.VMEM((b_per_w,), jnp.int32),
            pltpu.VMEM((_NSLOT, chunk, dim), jnp.float32),
            pltpu.SemaphoreType.DMA((_NSLOT,)),
            pltpu.SemaphoreType.DMA((_NSLOT,)),
        ],
        compiler_params=pltpu.CompilerParams(use_tc_tiling_on_sc=False),
    )
    def gather_kernel(table_hbm, idx_hbm, out_hbm, idx_v, rows_v, gsem, osem):
        wid = lax.axis_index("s") * nc + lax.axis_index("c")
        base = wid * b_per_w

        pltpu.sync_copy(idx_hbm.at[pl.ds(base, b_per_w)], idx_v)

        def gather_copy(i):
            s = i % _NSLOT
            return pltpu.make_async_copy(
                table_hbm.at[idx_v.at[pl.ds(i * chunk, chunk)]],
                rows_v.at[s],
                gsem.at[s],
            )

        def out_copy(i):
            s = i % _NSLOT
            return pltpu.make_async_copy(
                rows_v.at[s],
                out_hbm.at[pl.ds(base + i * chunk, chunk)],
                osem.at[s],
            )

        for i in range(min(_NGATHER, n_iters)):
            gather_copy(i).start()
        outs_pending = []
        for i in range(n_iters):
            gather_copy(i).wait()
            out_copy(i).start()
            outs_pending.append(i)
            nxt = i + _NGATHER
            if nxt < n_iters:
                reuse = nxt - _NSLOT
                if reuse >= 0:
                    out_copy(reuse).wait()
                    outs_pending.remove(reuse)
                gather_copy(nxt).start()
        for i in outs_pending:
            out_copy(i).wait()

    return gather_kernel


@jax.jit
def kernel(indices, feat_table):
    batch, hist = indices.shape
    vocab, dim = feat_table.shape
    n_rows = batch * hist
    idx_flat = indices.reshape(n_rows).astype(jnp.int32)
    out = _make_gather(n_rows, dim, chunk=800)(feat_table, idx_flat)
    return out.reshape(batch, hist, dim)
